# Initial kernel scaffold; baseline (speedup 1.0000x reference)
import jax
import jax.numpy as jnp
from jax.experimental import pallas as pl


def kernel(x, edge_index, edge_type, batch, weight1, comp1, root1, bias1, weight2, comp2, root2, bias2, W_clas, b_clas):
    raise NotImplementedError("write your pallas kernel here")


# speedup vs baseline: 4.1834x; 4.1834x over previous
import functools

import jax
import jax.numpy as jnp
from jax import lax
from jax.experimental import pallas as pl
from jax.experimental.pallas import tpu as pltpu
from jax.experimental.pallas import tpu_sc as plsc


_R = 4
_LANES = 16
_NW = 32
_NC = 2


def _cdiv(a, b):
    return (a + b - 1) // b


def _trans_body(x_ref, w_ref, comp_ref, root_ref, bias_ref, y_ref, s_ref):
    xb = x_ref[...]
    for r in range(_R):
        wr = comp_ref[r, 0] * w_ref[0]
        for b in range(1, _R):
            wr = wr + comp_ref[r, b] * w_ref[b]
        y_ref[:, r * 32:(r + 1) * 32] = jnp.dot(
            xb, wr, preferred_element_type=jnp.float32,
            precision=lax.Precision.HIGHEST)
    s_ref[...] = jnp.dot(
        xb, root_ref[...], preferred_element_type=jnp.float32,
            precision=lax.Precision.HIGHEST) + bias_ref[...]


def _trans_relu_body(sp_ref, ap_ref, w_ref, comp_ref, root_ref, bias_ref,
                     y_ref, s_ref):
    xb = jnp.maximum(sp_ref[...] + ap_ref[...], 0.0)
    for r in range(_R):
        wr = comp_ref[r, 0] * w_ref[0]
        for b in range(1, _R):
            wr = wr + comp_ref[r, b] * w_ref[b]
        y_ref[:, r * 32:(r + 1) * 32] = jnp.dot(
            xb, wr, preferred_element_type=jnp.float32,
            precision=lax.Precision.HIGHEST)
    s_ref[...] = jnp.dot(
        xb, root_ref[...], preferred_element_type=jnp.float32,
            precision=lax.Precision.HIGHEST) + bias_ref[...]


def _tc_transform(x_or_pair, weight, comp, root, bias, *, relu_in):
    n = (x_or_pair[0] if relu_in else x_or_pair).shape[0]
    bn = 2000
    grid = (n // bn,)
    full = lambda *shape: pl.BlockSpec(shape, lambda i: (0,) * len(shape))
    row_spec = pl.BlockSpec((bn, 32), lambda i: (i, 0))
    w_specs = [
        full(_R, 32, 32),
        pl.BlockSpec(memory_space=pltpu.SMEM),
        full(32, 32),
        full(1, 32),
    ]
    out_shape = (jax.ShapeDtypeStruct((n, 128), jnp.float32),
                 jax.ShapeDtypeStruct((n, 32), jnp.float32))
    out_specs = (pl.BlockSpec((bn, 128), lambda i: (i, 0)), row_spec)
    if relu_in:
        fn = pl.pallas_call(
            _trans_relu_body, grid=grid,
            in_specs=[row_spec, row_spec] + w_specs,
            out_specs=out_specs, out_shape=out_shape)
        return fn(x_or_pair[0], x_or_pair[1], weight, comp, root,
                  bias.reshape(1, 32))
    fn = pl.pallas_call(
        _trans_body, grid=grid,
        in_specs=[row_spec] + w_specs,
        out_specs=out_specs, out_shape=out_shape)
    return fn(x_or_pair, weight, comp, root, bias.reshape(1, 32))


def _winv_body(c0_ref, c1_ref, w_ref):
    w_ref[...] = 1.0 / jnp.maximum(c0_ref[...] + c1_ref[...], 1.0)


def _tc_winv(cnt):
    rows = cnt.shape[1]
    br = 320
    spec = pl.BlockSpec((br, 128), lambda i: (i, 0))
    fn = pl.pallas_call(
        _winv_body, grid=(rows // br,), in_specs=[spec, spec], out_specs=spec,
        out_shape=jax.ShapeDtypeStruct((rows, 128), jnp.float32))
    return fn(cnt[0], cnt[1])


def _final_body(p0_ref, p1_ref, c0_ref, c1_ref, wc_ref, bc_ref, out_ref):
    pooled = (p0_ref[...] + p1_ref[...]) / jnp.maximum(
        c0_ref[...] + c1_ref[...], 1.0)
    out_ref[...] = jnp.dot(
        pooled, wc_ref[...], preferred_element_type=jnp.float32,
            precision=lax.Precision.HIGHEST) + bc_ref[...]


def _tc_final(pool, pcnt, w_clas, b_clas):
    g = pool.shape[1]
    full = lambda *shape: pl.BlockSpec(shape, lambda: (0,) * len(shape))
    wc = jnp.pad(w_clas, ((0, 0), (0, 128 - w_clas.shape[1])))
    bc = jnp.pad(b_clas, (0, 128 - b_clas.shape[0])).reshape(1, 128)
    fn = pl.pallas_call(
        _final_body,
        in_specs=[full(g, 32), full(g, 32), full(g, 1), full(g, 1),
                  full(32, 128), full(1, 128)],
        out_specs=full(g, 128),
        out_shape=jax.ShapeDtypeStruct((g, 128), jnp.float32))
    out = fn(pool[0], pool[1], pcnt[0].reshape(g, 1), pcnt[1].reshape(g, 1),
             wc, bc)
    return out[:, :w_clas.shape[1]]


def _sc_count(sidx2, nr_pad):
    erows = sidx2.shape[0]
    rows_w = erows // _NW
    n_chunk = rows_w // 8
    per_tile = nr_pad // 16
    mesh = plsc.VectorSubcoreMesh(core_axis_name="c", subcore_axis_name="s")

    @functools.partial(
        pl.kernel,
        out_type=jax.ShapeDtypeStruct((_NC, nr_pad), jnp.float32),
        mesh=mesh,
        compiler_params=pltpu.CompilerParams(
            needs_layout_passes=False, use_tc_tiling_on_sc=False),
        scratch_types=[
            pltpu.VMEM((8, 128), jnp.int32),
            pltpu.VMEM((128,), jnp.float32),
            pltpu.VMEM((3200,), jnp.float32),
            pltpu.VMEM_SHARED((nr_pad,), jnp.float32),
            pltpu.SemaphoreType.DMA,
        ],
    )
    def count_kernel(sidx_hbm, out_hbm, idx_v, ones_v, zb_v, acc_sh, sem):
        cid = lax.axis_index("c")
        sid = lax.axis_index("s")
        wid = sid * _NC + cid

        def init_body(i, _):
            zb_v[pl.ds(i * 16, 16)] = jnp.zeros((16,), jnp.float32)
            return _
        lax.fori_loop(0, 200, init_body, 0)
        for gq in range(8):
            ones_v[pl.ds(gq * 16, 16)] = jnp.ones((16,), jnp.float32)
        for z in range(per_tile // 3200):
            pltpu.sync_copy(
                zb_v, acc_sh.at[pl.ds(sid * per_tile + z * 3200, 3200)])
        plsc.subcore_barrier()

        row0 = wid * rows_w

        def chunk_body(ch, _):
            pltpu.sync_copy(sidx_hbm.at[pl.ds(row0 + ch * 8, 8)], idx_v)
            for j in range(8):
                pltpu.sync_copy(ones_v, acc_sh.at[idx_v.at[j]], add=True)
            return _
        lax.fori_loop(0, n_chunk, chunk_body, 0)

        plsc.subcore_barrier()
        pltpu.sync_copy(acc_sh.at[pl.ds(sid * per_tile, per_tile)],
                        out_hbm.at[cid, pl.ds(sid * per_tile, per_tile)])

    return count_kernel(sidx2)


def _sc_edges(y_flat, gidx2, sidx2, w_flat, nh, n_pad):
    erows = gidx2.shape[0]
    rows_w = erows // 16
    cr = 4
    n_chunk = rows_w // cr
    acc_rows = nh + 2304
    zero_rows = acc_rows // 16
    out_tile = nh // 16
    mesh = plsc.VectorSubcoreMesh(core_axis_name="c", subcore_axis_name="s")

    @functools.partial(
        pl.kernel,
        out_type=jax.ShapeDtypeStruct((n_pad, 32), jnp.float32),
        mesh=mesh,
        compiler_params=pltpu.CompilerParams(
            needs_layout_passes=False, use_tc_tiling_on_sc=False),
        scratch_types=[
            pltpu.VMEM((cr, 128), jnp.int32),
            pltpu.VMEM((cr, 128), jnp.int32),
            pltpu.VMEM((cr, 128), jnp.int32),
            pltpu.VMEM((cr, 128), jnp.float32),
            pltpu.VMEM((cr * 128, 32), jnp.float32),
            pltpu.VMEM_SHARED((acc_rows, 32), jnp.float32),
            pltpu.SemaphoreType.DMA,
            pltpu.SemaphoreType.DMA,
        ],
    )
    def edge_kernel(y_hbm, gidx_hbm, sidx_hbm, w_hbm, out_hbm,
                    gi_v, si_v, sc_v, w_v, msg_v, acc_sh, sem_g, sem_w):
        cid = lax.axis_index("c")
        sid = lax.axis_index("s")
        base = cid * nh
        iota = lax.iota(jnp.int32, 16)

        def zinit(i, _):
            msg_v[i, pl.ds(0, 16)] = jnp.zeros((16,), jnp.float32)
            msg_v[i, pl.ds(16, 16)] = jnp.zeros((16,), jnp.float32)
            return _
        lax.fori_loop(0, cr * 128, zinit, 0)
        zrow0 = sid * zero_rows
        zoff = 0
        while zoff < zero_rows:
            nrows = min(cr * 128, zero_rows - zoff)
            pltpu.sync_copy(msg_v.at[pl.ds(0, nrows)],
                            acc_sh.at[pl.ds(zrow0 + zoff, nrows)])
            zoff += nrows
        plsc.subcore_barrier()

        row0 = sid * rows_w

        def chunk_body(ch, _):
            r0 = row0 + ch * cr
            pltpu.sync_copy(gidx_hbm.at[pl.ds(r0, cr)], gi_v)
            pltpu.sync_copy(sidx_hbm.at[pl.ds(r0, cr)], si_v)
            cps = [pltpu.async_copy(w_hbm.at[si_v.at[j]], w_v.at[j], sem_w)
                   for j in range(cr)]
            cpm = [pltpu.async_copy(y_hbm.at[gi_v.at[j]],
                                    msg_v.at[pl.ds(j * 128, 128)], sem_g)
                   for j in range(cr)]
            for cp in cps:
                cp.wait()
            for cp in cpm:
                cp.wait()

            def srow(s, _):
                for g in range(8):
                    sv = si_v[s, pl.ds(g * 16, 16)]
                    dloc = jnp.right_shift(sv, 2) - base
                    ok = (dloc >= 0) & (dloc < nh)
                    dump = (nh + ((ch & 3) * 512 + s * 128 + g * 16)) + iota
                    sc_v[s, pl.ds(g * 16, 16)] = jnp.where(ok, dloc, dump)
                    wv = w_v[s, pl.ds(g * 16, 16)]
                    riv = s * 128 + g * 16 + iota
                    for f in range(32):
                        civ = jnp.full((16,), f, jnp.int32)
                        mv = plsc.load_gather(msg_v, [riv, civ])
                        plsc.store_scatter(msg_v, [riv, civ], mv * wv)
                return _
            lax.fori_loop(0, cr, srow, 0)

            for j in range(cr):
                pltpu.sync_copy(msg_v.at[pl.ds(j * 128, 128)],
                                acc_sh.at[sc_v.at[j]], add=True)
            return _
        lax.fori_loop(0, n_chunk, chunk_body, 0)

        plsc.subcore_barrier()
        pltpu.sync_copy(
            acc_sh.at[pl.ds(sid * out_tile, out_tile)],
            out_hbm.at[pl.ds(base + sid * out_tile, out_tile)])

    return edge_kernel(y_flat, gidx2, sidx2, w_flat)


def _sc_pool(s2_pad, acc2, batch2, g_pad):
    n_pad = s2_pad.shape[0]
    rows_w = (n_pad // 128) // _NW
    per_tile = g_pad // 16
    mesh = plsc.VectorSubcoreMesh(core_axis_name="c", subcore_axis_name="s")

    @functools.partial(
        pl.kernel,
        out_type=(jax.ShapeDtypeStruct((_NC, 512, 32), jnp.float32),
                  jax.ShapeDtypeStruct((_NC, 512), jnp.float32)),
        mesh=mesh,
        compiler_params=pltpu.CompilerParams(
            needs_layout_passes=False, use_tc_tiling_on_sc=False),
        scratch_types=[
            pltpu.VMEM((1, 128), jnp.int32),
            pltpu.VMEM((128, 32), jnp.float32),
            pltpu.VMEM((128, 32), jnp.float32),
            pltpu.VMEM((128,), jnp.float32),
            pltpu.VMEM((per_tile, 32), jnp.float32),
            pltpu.VMEM((per_tile,), jnp.float32),
            pltpu.VMEM_SHARED((g_pad, 32), jnp.float32),
            pltpu.VMEM_SHARED((g_pad,), jnp.float32),
            pltpu.SemaphoreType.DMA,
        ],
    )
    def pool_kernel(s_hbm, a_hbm, b_hbm, pool_hbm, pcnt_hbm,
                    bi_v, s_v, h_v, ones_v, zb_v, zb1_v, pool_sh, cnt_sh,
                    sem):
        cid = lax.axis_index("c")
        sid = lax.axis_index("s")
        wid = sid * _NC + cid

        def zinit(i, _):
            zb_v[i, pl.ds(0, 16)] = jnp.zeros((16,), jnp.float32)
            zb_v[i, pl.ds(16, 16)] = jnp.zeros((16,), jnp.float32)
            return _
        lax.fori_loop(0, per_tile, zinit, 0)
        for gq in range(per_tile // 16):
            zb1_v[pl.ds(gq * 16, 16)] = jnp.zeros((16,), jnp.float32)
        for gq in range(8):
            ones_v[pl.ds(gq * 16, 16)] = jnp.ones((16,), jnp.float32)
        pltpu.sync_copy(zb_v, pool_sh.at[pl.ds(sid * per_tile, per_tile)])
        pltpu.sync_copy(zb1_v, cnt_sh.at[pl.ds(sid * per_tile, per_tile)])
        plsc.subcore_barrier()

        row0 = wid * rows_w

        def chunk_body(ch, _):
            r = row0 + ch
            pltpu.sync_copy(b_hbm.at[pl.ds(r, 1)], bi_v)
            pltpu.sync_copy(s_hbm.at[pl.ds(r * 128, 128)], s_v)
            pltpu.sync_copy(a_hbm.at[pl.ds(r * 128, 128)], h_v)

            def hrow(t, _):
                h_v[t, pl.ds(0, 16)] = jnp.maximum(
                    h_v[t, pl.ds(0, 16)] + s_v[t, pl.ds(0, 16)], 0.0)
                h_v[t, pl.ds(16, 16)] = jnp.maximum(
                    h_v[t, pl.ds(16, 16)] + s_v[t, pl.ds(16, 16)], 0.0)
                return _
            lax.fori_loop(0, 128, hrow, 0)

            pltpu.sync_copy(h_v, pool_sh.at[bi_v.at[0]], add=True)
            pltpu.sync_copy(ones_v, cnt_sh.at[bi_v.at[0]], add=True)
            return _
        lax.fori_loop(0, rows_w, chunk_body, 0)

        plsc.subcore_barrier()
        out_rows = 512 // 16
        pltpu.sync_copy(pool_sh.at[pl.ds(sid * out_rows, out_rows)],
                        pool_hbm.at[cid, pl.ds(sid * out_rows, out_rows)])
        pltpu.sync_copy(cnt_sh.at[pl.ds(sid * out_rows, out_rows)],
                        pcnt_hbm.at[cid, pl.ds(sid * out_rows, out_rows)])

    return pool_kernel(s2_pad, acc2, batch2)


def kernel(x, edge_index, edge_type, batch,
           weight1, comp1, root1, bias1,
           weight2, comp2, root2, bias2,
           W_clas, b_clas):
    n = x.shape[0]
    e = edge_index.shape[1]
    nh = 51200
    n_pad = 2 * nh
    nr = n * _R
    nr_pad = 409600
    e_pad = 1638400

    src = edge_index[0]
    dst = edge_index[1]
    gidx = src * _R + edge_type
    sidx = dst * _R + edge_type
    pad = nr + (jnp.arange(e_pad - e, dtype=jnp.int32) % (nr_pad - nr))
    gidx2 = jnp.concatenate(
        [gidx, jnp.arange(e_pad - e, dtype=jnp.int32) % nr]).reshape(-1, 128)
    sidx2 = jnp.concatenate([sidx, pad]).reshape(-1, 128)

    cnt = _sc_count(sidx2, nr_pad)
    w_flat = _tc_winv(cnt.reshape(2, -1, 128)).reshape(-1)

    y1, s1 = _tc_transform(x, weight1, comp1, root1, bias1, relu_in=False)
    acc1 = _sc_edges(y1.reshape(nr, 32), gidx2, sidx2, w_flat, nh, n_pad)

    y2, s2 = _tc_transform((s1, acc1[:n]), weight2, comp2, root2, bias2,
                           relu_in=True)
    acc2 = _sc_edges(y2.reshape(nr, 32), gidx2, sidx2, w_flat, nh, n_pad)

    g_pad = 1024
    s2_pad = jnp.pad(s2, ((0, n_pad - n), (0, 0)))
    batch2 = jnp.pad(batch, (0, n_pad - n), constant_values=512).reshape(
        -1, 128)
    pool, pcnt = _sc_pool(s2_pad, acc2, batch2, g_pad)

    return _tc_final(pool, pcnt, W_clas, b_clas)


# speedup vs baseline: 4.7440x; 1.1340x over previous
import functools

import jax
import jax.numpy as jnp
from jax import lax
from jax.experimental import pallas as pl
from jax.experimental.pallas import tpu as pltpu
from jax.experimental.pallas import tpu_sc as plsc


_R = 4
_LANES = 16
_NW = 32
_NC = 2


def _cdiv(a, b):
    return (a + b - 1) // b


def _trans_body(x_ref, w_ref, comp_ref, root_ref, bias_ref, y_ref, s_ref):
    xb = x_ref[...]
    for r in range(_R):
        wr = comp_ref[r, 0] * w_ref[0]
        for b in range(1, _R):
            wr = wr + comp_ref[r, b] * w_ref[b]
        y_ref[:, r * 32:(r + 1) * 32] = jnp.dot(
            xb, wr, preferred_element_type=jnp.float32,
            precision=lax.Precision.HIGHEST)
    s_ref[...] = jnp.dot(
        xb, root_ref[...], preferred_element_type=jnp.float32,
            precision=lax.Precision.HIGHEST) + bias_ref[...]


def _trans_relu_body(sp_ref, ap_ref, w_ref, comp_ref, root_ref, bias_ref,
                     y_ref, s_ref):
    xb = jnp.maximum(sp_ref[...] + ap_ref[...], 0.0)
    for r in range(_R):
        wr = comp_ref[r, 0] * w_ref[0]
        for b in range(1, _R):
            wr = wr + comp_ref[r, b] * w_ref[b]
        y_ref[:, r * 32:(r + 1) * 32] = jnp.dot(
            xb, wr, preferred_element_type=jnp.float32,
            precision=lax.Precision.HIGHEST)
    s_ref[...] = jnp.dot(
        xb, root_ref[...], preferred_element_type=jnp.float32,
            precision=lax.Precision.HIGHEST) + bias_ref[...]


def _tc_transform(x_or_pair, weight, comp, root, bias, *, relu_in):
    n = (x_or_pair[0] if relu_in else x_or_pair).shape[0]
    bn = 2000
    grid = (n // bn,)
    full = lambda *shape: pl.BlockSpec(shape, lambda i: (0,) * len(shape))
    row_spec = pl.BlockSpec((bn, 32), lambda i: (i, 0))
    w_specs = [
        full(_R, 32, 32),
        pl.BlockSpec(memory_space=pltpu.SMEM),
        full(32, 32),
        full(1, 32),
    ]
    out_shape = (jax.ShapeDtypeStruct((n, 128), jnp.float32),
                 jax.ShapeDtypeStruct((n, 32), jnp.float32))
    out_specs = (pl.BlockSpec((bn, 128), lambda i: (i, 0)), row_spec)
    if relu_in:
        fn = pl.pallas_call(
            _trans_relu_body, grid=grid,
            in_specs=[row_spec, row_spec] + w_specs,
            out_specs=out_specs, out_shape=out_shape)
        return fn(x_or_pair[0], x_or_pair[1], weight, comp, root,
                  bias.reshape(1, 32))
    fn = pl.pallas_call(
        _trans_body, grid=grid,
        in_specs=[row_spec] + w_specs,
        out_specs=out_specs, out_shape=out_shape)
    return fn(x_or_pair, weight, comp, root, bias.reshape(1, 32))


def _winv_body(c0_ref, c1_ref, w_ref):
    w_ref[...] = 1.0 / jnp.maximum(c0_ref[...] + c1_ref[...], 1.0)


def _tc_winv(cnt):
    rows = cnt.shape[1]
    br = 320
    spec = pl.BlockSpec((br, 128), lambda i: (i, 0))
    fn = pl.pallas_call(
        _winv_body, grid=(rows // br,), in_specs=[spec, spec], out_specs=spec,
        out_shape=jax.ShapeDtypeStruct((rows, 128), jnp.float32))
    return fn(cnt[0], cnt[1])


def _final_body(p0_ref, p1_ref, c0_ref, c1_ref, wc_ref, bc_ref, out_ref):
    pooled = (p0_ref[...] + p1_ref[...]) / jnp.maximum(
        c0_ref[...] + c1_ref[...], 1.0)
    out_ref[...] = jnp.dot(
        pooled, wc_ref[...], preferred_element_type=jnp.float32,
            precision=lax.Precision.HIGHEST) + bc_ref[...]


def _tc_final(pool, pcnt, w_clas, b_clas):
    g = pool.shape[1]
    full = lambda *shape: pl.BlockSpec(shape, lambda: (0,) * len(shape))
    wc = jnp.pad(w_clas, ((0, 0), (0, 128 - w_clas.shape[1])))
    bc = jnp.pad(b_clas, (0, 128 - b_clas.shape[0])).reshape(1, 128)
    fn = pl.pallas_call(
        _final_body,
        in_specs=[full(g, 32), full(g, 32), full(g, 1), full(g, 1),
                  full(32, 128), full(1, 128)],
        out_specs=full(g, 128),
        out_shape=jax.ShapeDtypeStruct((g, 128), jnp.float32))
    out = fn(pool[0], pool[1], pcnt[0].reshape(g, 1), pcnt[1].reshape(g, 1),
             wc, bc)
    return out[:, :w_clas.shape[1]]


def _sc_count(sidx2, nr_pad):
    erows = sidx2.shape[0]
    rows_w = erows // _NW
    n_chunk = rows_w // 8
    per_tile = nr_pad // 16
    mesh = plsc.VectorSubcoreMesh(core_axis_name="c", subcore_axis_name="s")

    @functools.partial(
        pl.kernel,
        out_type=jax.ShapeDtypeStruct((_NC, nr_pad), jnp.float32),
        mesh=mesh,
        compiler_params=pltpu.CompilerParams(
            needs_layout_passes=False, use_tc_tiling_on_sc=False),
        scratch_types=[
            pltpu.VMEM((8, 128), jnp.int32),
            pltpu.VMEM((128,), jnp.float32),
            pltpu.VMEM((3200,), jnp.float32),
            pltpu.VMEM_SHARED((nr_pad,), jnp.float32),
            pltpu.SemaphoreType.DMA,
        ],
    )
    def count_kernel(sidx_hbm, out_hbm, idx_v, ones_v, zb_v, acc_sh, sem):
        cid = lax.axis_index("c")
        sid = lax.axis_index("s")
        wid = sid * _NC + cid

        def init_body(i, _):
            zb_v[pl.ds(i * 16, 16)] = jnp.zeros((16,), jnp.float32)
            return _
        lax.fori_loop(0, 200, init_body, 0)
        for gq in range(8):
            ones_v[pl.ds(gq * 16, 16)] = jnp.ones((16,), jnp.float32)
        for z in range(per_tile // 3200):
            pltpu.sync_copy(
                zb_v, acc_sh.at[pl.ds(sid * per_tile + z * 3200, 3200)])
        plsc.subcore_barrier()

        row0 = wid * rows_w

        def chunk_body(ch, _):
            pltpu.sync_copy(sidx_hbm.at[pl.ds(row0 + ch * 8, 8)], idx_v)
            for j in range(8):
                pltpu.sync_copy(ones_v, acc_sh.at[idx_v.at[j]], add=True)
            return _
        lax.fori_loop(0, n_chunk, chunk_body, 0)

        plsc.subcore_barrier()
        pltpu.sync_copy(acc_sh.at[pl.ds(sid * per_tile, per_tile)],
                        out_hbm.at[cid, pl.ds(sid * per_tile, per_tile)])

    return count_kernel(sidx2)


def _sc_edges(y_flat, gidx2, sidx2, w_flat, nh, n_pad):
    erows = gidx2.shape[0]
    rows_w = erows // 16
    nbuf = 4
    acc_rows = nh + 256
    zero_rows = acc_rows // 16
    out_tile = nh // 16
    mesh = plsc.VectorSubcoreMesh(core_axis_name="c", subcore_axis_name="s")

    @functools.partial(
        pl.kernel,
        out_type=jax.ShapeDtypeStruct((n_pad, 32), jnp.float32),
        mesh=mesh,
        compiler_params=pltpu.CompilerParams(
            needs_layout_passes=False, use_tc_tiling_on_sc=False),
        scratch_types=[
            pltpu.VMEM((nbuf, 128), jnp.int32),
            pltpu.VMEM((nbuf, 128), jnp.int32),
            pltpu.VMEM((nbuf, 128), jnp.int32),
            pltpu.VMEM((nbuf, 128), jnp.float32),
            pltpu.VMEM((nbuf * 128, 32), jnp.float32),
            pltpu.VMEM_SHARED((acc_rows, 32), jnp.float32),
            [pltpu.SemaphoreType.DMA] * nbuf,
            [pltpu.SemaphoreType.DMA] * nbuf,
            [pltpu.SemaphoreType.DMA] * nbuf,
            [pltpu.SemaphoreType.DMA] * nbuf,
        ],
    )
    def edge_kernel(y_hbm, gidx_hbm, sidx_hbm, w_hbm, out_hbm,
                    gi_v, si_v, sc_v, w_v, msg_v, acc_sh,
                    sem_i, sem_g, sem_w, sem_s):
        cid = lax.axis_index("c")
        sid = lax.axis_index("s")
        base = cid * nh
        iota = lax.iota(jnp.int32, 16)

        def zinit(i, _):
            msg_v[i, pl.ds(0, 16)] = jnp.zeros((16,), jnp.float32)
            msg_v[i, pl.ds(16, 16)] = jnp.zeros((16,), jnp.float32)
            return _
        lax.fori_loop(0, nbuf * 128, zinit, 0)
        zrow0 = sid * zero_rows
        zoff = 0
        while zoff < zero_rows:
            nrows = min(nbuf * 128, zero_rows - zoff)
            pltpu.sync_copy(msg_v.at[pl.ds(0, nrows)],
                            acc_sh.at[pl.ds(zrow0 + zoff, nrows)])
            zoff += nrows
        plsc.subcore_barrier()

        row0 = sid * rows_w

        def slot_dispatch(u, fn):
            b = lax.rem(u, nbuf)
            for sb in range(nbuf):
                @pl.when(b == sb)
                def _():
                    fn(sb)

        def issue_idx(u):
            def go(sb):
                pltpu.async_copy(gidx_hbm.at[pl.ds(row0 + u, 1)],
                                 gi_v.at[pl.ds(sb, 1)], sem_i[sb])
                pltpu.async_copy(sidx_hbm.at[pl.ds(row0 + u, 1)],
                                 si_v.at[pl.ds(sb, 1)], sem_i[sb])
            slot_dispatch(u, go)

        def wait_idx(u):
            def go(sb):
                pltpu.make_async_copy(
                    gidx_hbm.at[pl.ds(row0 + u, 1)],
                    gi_v.at[pl.ds(sb, 1)], sem_i[sb]).wait()
                pltpu.make_async_copy(
                    sidx_hbm.at[pl.ds(row0 + u, 1)],
                    si_v.at[pl.ds(sb, 1)], sem_i[sb]).wait()
            slot_dispatch(u, go)

        def gathers(u, do_issue):
            def go(sb):
                if do_issue:
                    pltpu.async_copy(y_hbm.at[gi_v.at[sb]],
                                     msg_v.at[pl.ds(sb * 128, 128)],
                                     sem_g[sb])
                    pltpu.async_copy(w_hbm.at[si_v.at[sb]], w_v.at[sb],
                                     sem_w[sb])
                else:
                    pltpu.make_async_copy(
                        y_hbm.at[gi_v.at[sb]],
                        msg_v.at[pl.ds(sb * 128, 128)], sem_g[sb]).wait()
                    pltpu.make_async_copy(
                        w_hbm.at[si_v.at[sb]], w_v.at[sb], sem_w[sb]).wait()
            slot_dispatch(u, go)

        def scatter(u, do_issue):
            def go(sb):
                if do_issue:
                    pltpu.async_copy(msg_v.at[pl.ds(sb * 128, 128)],
                                     acc_sh.at[sc_v.at[sb]], sem_s[sb],
                                     add=True)
                else:
                    pltpu.make_async_copy(
                        msg_v.at[pl.ds(sb * 128, 128)],
                        acc_sh.at[sc_v.at[sb]], sem_s[sb]).wait()
            slot_dispatch(u, go)

        def compute(u):
            b = lax.rem(u, nbuf)
            for g in range(8):
                sv = si_v[b, pl.ds(g * 16, 16)]
                dloc = jnp.right_shift(sv, 2) - base
                ok = (dloc >= 0) & (dloc < nh)
                dump = (nh + ((u & 1) * 128 + g * 16)) + iota
                sc_v[b, pl.ds(g * 16, 16)] = jnp.where(ok, dloc, dump)
                wv = w_v[b, pl.ds(g * 16, 16)]
                riv = b * 128 + g * 16 + iota
                for f in range(32):
                    civ = jnp.full((16,), f, jnp.int32)
                    mv = plsc.load_gather(msg_v, [riv, civ])
                    plsc.store_scatter(msg_v, [riv, civ], mv * wv)

        for u in range(nbuf):
            issue_idx(u)
        for u in range(2):
            wait_idx(u)
            gathers(u, True)

        def unit_body(u, carry):
            @pl.when(u >= 2)
            def _():
                scatter(u - 2, False)
            @pl.when(u + 2 < rows_w)
            def _():
                wait_idx(u + 2)
                gathers(u + 2, True)
            @pl.when(u < rows_w)
            def _():
                gathers(u, False)
                compute(u)
                scatter(u, True)
            @pl.when(u + nbuf < rows_w)
            def _():
                issue_idx(u + nbuf)
            return carry
        lax.fori_loop(0, rows_w + 2, unit_body, 0)

        plsc.subcore_barrier()
        pltpu.sync_copy(
            acc_sh.at[pl.ds(sid * out_tile, out_tile)],
            out_hbm.at[pl.ds(base + sid * out_tile, out_tile)])

    return edge_kernel(y_flat, gidx2, sidx2, w_flat)


def _sc_pool(s2_pad, acc2, batch2, g_pad):
    n_pad = s2_pad.shape[0]
    rows_w = (n_pad // 128) // _NW
    per_tile = g_pad // 16
    mesh = plsc.VectorSubcoreMesh(core_axis_name="c", subcore_axis_name="s")

    @functools.partial(
        pl.kernel,
        out_type=(jax.ShapeDtypeStruct((_NC, 512, 32), jnp.float32),
                  jax.ShapeDtypeStruct((_NC, 512), jnp.float32)),
        mesh=mesh,
        compiler_params=pltpu.CompilerParams(
            needs_layout_passes=False, use_tc_tiling_on_sc=False),
        scratch_types=[
            pltpu.VMEM((1, 128), jnp.int32),
            pltpu.VMEM((128, 32), jnp.float32),
            pltpu.VMEM((128, 32), jnp.float32),
            pltpu.VMEM((128,), jnp.float32),
            pltpu.VMEM((per_tile, 32), jnp.float32),
            pltpu.VMEM((per_tile,), jnp.float32),
            pltpu.VMEM_SHARED((g_pad, 32), jnp.float32),
            pltpu.VMEM_SHARED((g_pad,), jnp.float32),
            pltpu.SemaphoreType.DMA,
        ],
    )
    def pool_kernel(s_hbm, a_hbm, b_hbm, pool_hbm, pcnt_hbm,
                    bi_v, s_v, h_v, ones_v, zb_v, zb1_v, pool_sh, cnt_sh,
                    sem):
        cid = lax.axis_index("c")
        sid = lax.axis_index("s")
        wid = sid * _NC + cid

        def zinit(i, _):
            zb_v[i, pl.ds(0, 16)] = jnp.zeros((16,), jnp.float32)
            zb_v[i, pl.ds(16, 16)] = jnp.zeros((16,), jnp.float32)
            return _
        lax.fori_loop(0, per_tile, zinit, 0)
        for gq in range(per_tile // 16):
            zb1_v[pl.ds(gq * 16, 16)] = jnp.zeros((16,), jnp.float32)
        for gq in range(8):
            ones_v[pl.ds(gq * 16, 16)] = jnp.ones((16,), jnp.float32)
        pltpu.sync_copy(zb_v, pool_sh.at[pl.ds(sid * per_tile, per_tile)])
        pltpu.sync_copy(zb1_v, cnt_sh.at[pl.ds(sid * per_tile, per_tile)])
        plsc.subcore_barrier()

        row0 = wid * rows_w

        def chunk_body(ch, _):
            r = row0 + ch
            pltpu.sync_copy(b_hbm.at[pl.ds(r, 1)], bi_v)
            pltpu.sync_copy(s_hbm.at[pl.ds(r * 128, 128)], s_v)
            pltpu.sync_copy(a_hbm.at[pl.ds(r * 128, 128)], h_v)

            def hrow(t, _):
                h_v[t, pl.ds(0, 16)] = jnp.maximum(
                    h_v[t, pl.ds(0, 16)] + s_v[t, pl.ds(0, 16)], 0.0)
                h_v[t, pl.ds(16, 16)] = jnp.maximum(
                    h_v[t, pl.ds(16, 16)] + s_v[t, pl.ds(16, 16)], 0.0)
                return _
            lax.fori_loop(0, 128, hrow, 0)

            pltpu.sync_copy(h_v, pool_sh.at[bi_v.at[0]], add=True)
            pltpu.sync_copy(ones_v, cnt_sh.at[bi_v.at[0]], add=True)
            return _
        lax.fori_loop(0, rows_w, chunk_body, 0)

        plsc.subcore_barrier()
        out_rows = 512 // 16
        pltpu.sync_copy(pool_sh.at[pl.ds(sid * out_rows, out_rows)],
                        pool_hbm.at[cid, pl.ds(sid * out_rows, out_rows)])
        pltpu.sync_copy(cnt_sh.at[pl.ds(sid * out_rows, out_rows)],
                        pcnt_hbm.at[cid, pl.ds(sid * out_rows, out_rows)])

    return pool_kernel(s2_pad, acc2, batch2)


def kernel(x, edge_index, edge_type, batch,
           weight1, comp1, root1, bias1,
           weight2, comp2, root2, bias2,
           W_clas, b_clas):
    n = x.shape[0]
    e = edge_index.shape[1]
    nh = 51200
    n_pad = 2 * nh
    nr = n * _R
    nr_pad = 409600
    e_pad = 1638400

    src = edge_index[0]
    dst = edge_index[1]
    gidx = src * _R + edge_type
    sidx = dst * _R + edge_type
    pad = nr + (jnp.arange(e_pad - e, dtype=jnp.int32) % (nr_pad - nr))
    gidx2 = jnp.concatenate(
        [gidx, jnp.arange(e_pad - e, dtype=jnp.int32) % nr]).reshape(-1, 128)
    sidx2 = jnp.concatenate([sidx, pad]).reshape(-1, 128)

    cnt = _sc_count(sidx2, nr_pad)
    w_flat = _tc_winv(cnt.reshape(2, -1, 128)).reshape(-1)

    y1, s1 = _tc_transform(x, weight1, comp1, root1, bias1, relu_in=False)
    acc1 = _sc_edges(y1.reshape(nr, 32), gidx2, sidx2, w_flat, nh, n_pad)

    y2, s2 = _tc_transform((s1, acc1[:n]), weight2, comp2, root2, bias2,
                           relu_in=True)
    acc2 = _sc_edges(y2.reshape(nr, 32), gidx2, sidx2, w_flat, nh, n_pad)

    g_pad = 1024
    s2_pad = jnp.pad(s2, ((0, n_pad - n), (0, 0)))
    batch2 = jnp.pad(batch, (0, n_pad - n), constant_values=512).reshape(
        -1, 128)
    pool, pcnt = _sc_pool(s2_pad, acc2, batch2, g_pad)

    return _tc_final(pool, pcnt, W_clas, b_clas)
